# Initial kernel scaffold; baseline (speedup 1.0000x reference)
import jax
import jax.numpy as jnp
from jax.experimental import pallas as pl


def kernel(all_embed, relation_emb, inter_edge_w, edge_index, edge_type, inter_edge, users, pos_items, neg_items):
    raise NotImplementedError("write your pallas kernel here")


# speedup vs baseline: 1.1757x; 1.1757x over previous
import functools

import jax
import jax.numpy as jnp
from jax import lax
from jax.experimental import pallas as pl
from jax.experimental.pallas import tpu as pltpu
from jax.experimental.pallas import tpu_sc as plsc

F32 = jnp.float32
I32 = jnp.int32

NU, NI, NENT = 30000, 20000, 50000
D = 64
E, EUI, BATCH, TOPK = 800_000, 300_000, 1024, 256
NC, NS, LN = 2, 16, 16
NW = NC * NS
EP = 819_200
ER = EP // 128
EPT = EP // NW
ROWS_T = EPT // 128
UIP = 311_296
UIR = UIP // 128
DENW = 51_200

_mesh = plsc.VectorSubcoreMesh(
    core_axis_name="c", subcore_axis_name="s", num_cores=NC, num_subcores=NS)


def _it():
    return lax.iota(I32, LN)


def _wid():
    return lax.axis_index("c") * NS + lax.axis_index("s")


@functools.partial(
    pl.kernel,
    out_type=[
        jax.ShapeDtypeStruct((ER, 128), F32),
        jax.ShapeDtypeStruct((NC * DENW,), F32),
    ],
    mesh=_mesh,
    compiler_params=pltpu.CompilerParams(
        needs_layout_passes=False, use_tc_tiling_on_sc=False),
    scratch_types=dict(
        rel_v=pltpu.VMEM((576,), F32),
        hb=pltpu.VMEM((4, 128), I32),
        tb=pltpu.VMEM((4, 128), I32),
        yb=pltpu.VMEM((4, 128), I32),
        hr=pltpu.VMEM((512, D), F32),
        tr=pltpu.VMEM((512, D), F32),
        exb=pltpu.VMEM((4, 128), F32),
        zb=pltpu.VMEM((3200,), F32),
        den_sh=pltpu.VMEM_SHARED((DENW,), F32),
        s1=pltpu.SemaphoreType.DMA,
        s2=pltpu.SemaphoreType.DMA,
    ),
)
def _k1(ent, relf, head2, tail2, typ2, ex2, den_p, rel_v, hb, tb, yb, hr, tr,
        exb, zb, den_sh, s1, s2):
    c = lax.axis_index("c")
    s = lax.axis_index("s")
    wid = c * NS + s
    it = _it()
    zf = jnp.zeros((LN,), F32)

    @pl.loop(0, 200)
    def _z(i):
        zb[pl.ds(i * 16, 16)] = zf

    pltpu.sync_copy(zb, den_sh.at[pl.ds(s * 3200, 3200)])
    pltpu.sync_copy(relf, rel_v)
    plsc.subcore_barrier()

    base_row = wid * ROWS_T

    @pl.loop(0, 50)
    def _chunk(ci):
        r0 = base_row + ci * 4
        pltpu.sync_copy(head2.at[pl.ds(r0, 4)], hb)
        pltpu.sync_copy(tail2.at[pl.ds(r0, 4)], tb)
        pltpu.sync_copy(typ2.at[pl.ds(r0, 4)], yb)
        cps = []
        for j in range(4):
            cps.append(pltpu.async_copy(
                ent.at[hb.at[j]], hr.at[pl.ds(j * 128, 128)], s1))
            cps.append(pltpu.async_copy(
                ent.at[tb.at[j]], tr.at[pl.ds(j * 128, 128)], s2))
        for cp in cps:
            cp.wait()

        @pl.loop(0, 32)
        def _grp(g):
            pos = g * 16 + it
            rr = pos >> 7
            cc = pos & 127
            tyv = plsc.load_gather(yb, [rr, cc])
            relbase = (tyv - 1) * 64
            acc = zf
            for d in range(D):
                dv = jnp.full((LN,), d, I32)
                h = plsc.load_gather(hr, [pos, dv])
                t = plsc.load_gather(tr, [pos, dv])
                r = plsc.load_gather(rel_v, [relbase + d])
                acc = acc + h * t * r
            exv = jnp.exp(acc * 0.125)
            gid = wid * EPT + ci * 512 + pos
            exm = jnp.where(gid < E, exv, 0.0)
            plsc.store_scatter(exb, [rr, cc], exm)

        for j in range(4):
            pltpu.sync_copy(exb.at[j], den_sh.at[hb.at[j]], add=True)
        pltpu.sync_copy(exb, ex2.at[pl.ds(r0, 4)])

    plsc.subcore_barrier()

    @pl.when(s == 0)
    def _out():
        pltpu.sync_copy(den_sh, den_p.at[pl.ds(c * DENW, DENW)])


def _radix_select(buf, ngroups, k, hist, hsum):
    it = _it()
    prefix = jnp.int32(0)
    rank = jnp.int32(k)
    for rnd in range(4):
        shift = 24 - 8 * rnd

        @pl.loop(0, 256)
        def _z(i):
            hist[pl.ds(i * 16, 16)] = jnp.zeros((LN,), I32)

        def _hbody(i, carry):
            v = buf[pl.ds(i * 16, 16)]
            u = plsc.bitcast(v, I32)
            inset = v >= 0.0
            if rnd > 0:
                inset = inset & ((u >> (shift + 8)) == carry)
            bin_ = (u >> shift) & 255
            idxv = bin_ * 16 + it
            a = plsc.load_gather(hist, [idxv], mask=inset)
            plsc.store_scatter(hist, [idxv], a + 1, mask=inset)
            return carry

        lax.fori_loop(0, ngroups, _hbody, prefix)

        @pl.loop(0, 16)
        def _s(i):
            acc = jnp.zeros((LN,), I32)
            base = (i * 16 + it) * 16
            for l in range(16):
                acc = acc + plsc.load_gather(hist, [base + l])
            hsum[pl.ds(i * 16, 16)] = acc

        def _gbody(g, st):
            run, bstar, above = st
            gi = 15 - g
            h = hsum[pl.ds(gi * 16, 16)]
            hr_ = lax.rev(h, (0,))
            cs = plsc.cumsum(hr_)
            tot = run + cs
            hit = tot >= rank
            anyhit = jnp.max(jnp.where(hit, 1, 0)) > 0
            lane = jnp.max(plsc.all_reduce_ffs(hit))
            lm = it == lane
            sb = jnp.max(jnp.where(lm, tot, 0))
            hbv = jnp.max(jnp.where(lm, hr_, 0))
            found = anyhit & (bstar < 0)
            bstar = jnp.where(found, gi * 16 + 15 - lane, bstar)
            above = jnp.where(found, sb - hbv, above)
            run = jnp.max(tot)
            return run, bstar, above

        _, bstar, above = lax.fori_loop(
            0, 16, _gbody, (jnp.int32(0), jnp.int32(-1), jnp.int32(0)))
        rank = rank - above
        prefix = (prefix << 8) | bstar
    return prefix, rank


def _select_topk(buf, idx_src, ngroups, gbase, vbits, m, gt_v, gt_i, eq_i,
                 out_v, out_i):
    it = _it()
    vstar = plsc.bitcast(jnp.full((LN,), vbits, I32), F32)
    cgt = TOPK - m

    def _sel(i, st):
        ogt, oeq = st
        v = buf[pl.ds(i * 16, 16)]
        if idx_src is None:
            gid = gbase + i * 16 + it
        else:
            gid = idx_src[pl.ds(i * 16, 16)]
        gt = v > vstar
        plsc.store_compressed(gt_v.at[pl.ds(ogt, 16)], v, mask=gt)
        plsc.store_compressed(gt_i.at[pl.ds(ogt, 16)], gid, mask=gt)
        eq = (v == vstar) & (oeq < 256)
        plsc.store_compressed(eq_i.at[pl.ds(oeq, 16)], gid, mask=eq)
        ogt = ogt + jnp.max(plsc.all_reduce_population_count(gt))
        oeq = oeq + jnp.max(plsc.all_reduce_population_count(eq))
        return ogt, oeq

    lax.fori_loop(0, ngroups, _sel, (jnp.int32(0), jnp.int32(0)))
    for i in range(16):
        pos = i * 16 + it
        ingt = pos < cgt
        gv = plsc.load_gather(gt_v, [jnp.where(ingt, pos, 0)])
        gi_ = plsc.load_gather(gt_i, [jnp.where(ingt, pos, 0)])
        ei = plsc.load_gather(eq_i, [jnp.where(ingt, 0, pos - cgt)])
        out_v[pl.ds(i * 16, 16)] = jnp.where(ingt, gv, vstar)
        out_i[pl.ds(i * 16, 16)] = jnp.where(ingt, gi_, ei)


@functools.partial(
    pl.kernel,
    out_type=[
        jax.ShapeDtypeStruct((ER, 128), F32),
        jax.ShapeDtypeStruct((NW * 256,), F32),
        jax.ShapeDtypeStruct((NW * 256,), I32),
    ],
    mesh=_mesh,
    compiler_params=pltpu.CompilerParams(
        needs_layout_passes=False, use_tc_tiling_on_sc=False),
    scratch_types=dict(
        den_v=pltpu.VMEM((DENW,), F32),
        tmp_v=pltpu.VMEM((6400,), F32),
        exb=pltpu.VMEM((4, 128), F32),
        hb=pltpu.VMEM((4, 128), I32),
        attn_v=pltpu.VMEM((EPT,), F32),
        hist=pltpu.VMEM((4096,), I32),
        hsum=pltpu.VMEM((256,), I32),
        gt_v=pltpu.VMEM((272,), F32),
        gt_i=pltpu.VMEM((272,), I32),
        eq_i=pltpu.VMEM((272,), I32),
        out_v=pltpu.VMEM((256,), F32),
        out_i=pltpu.VMEM((256,), I32),
    ),
)
def _k2(ex2, head2, den_p, attn2, candv, candi, den_v, tmp_v, exb, hb, attn_v,
        hist, hsum, gt_v, gt_i, eq_i, out_v, out_i):
    wid = _wid()
    it = _it()
    pltpu.sync_copy(den_p.at[pl.ds(0, DENW)], den_v)

    @pl.loop(0, 8)
    def _dq(q):
        pltpu.sync_copy(den_p.at[pl.ds(DENW + q * 6400, 6400)], tmp_v)

        @pl.loop(0, 400)
        def _da(i):
            o = q * 6400 + i * 16
            den_v[pl.ds(o, 16)] = den_v[pl.ds(o, 16)] + tmp_v[pl.ds(i * 16, 16)]

    base_row = wid * ROWS_T

    @pl.loop(0, 50)
    def _chunk(ci):
        r0 = base_row + ci * 4
        pltpu.sync_copy(ex2.at[pl.ds(r0, 4)], exb)
        pltpu.sync_copy(head2.at[pl.ds(r0, 4)], hb)

        @pl.loop(0, 32)
        def _g(g):
            pos = g * 16 + it
            rr = pos >> 7
            cc = pos & 127
            exv = plsc.load_gather(exb, [rr, cc])
            hv = plsc.load_gather(hb, [rr, cc])
            dn = plsc.load_gather(den_v, [hv])
            at = exv / (dn + 1e-16)
            gid = wid * EPT + ci * 512 + pos
            valid = gid < E
            plsc.store_scatter(exb, [rr, cc], jnp.where(valid, at, 0.0))
            attn_v[pl.ds(ci * 512 + g * 16, 16)] = jnp.where(valid, at, -1.0)

        pltpu.sync_copy(exb, attn2.at[pl.ds(r0, 4)])

    vbits, m = _radix_select(attn_v, EPT // 16, TOPK, hist, hsum)
    _select_topk(attn_v, None, EPT // 16, wid * EPT, vbits, m, gt_v, gt_i,
                 eq_i, out_v, out_i)
    pltpu.sync_copy(out_v, candv.at[pl.ds(wid * 256, 256)])
    pltpu.sync_copy(out_i, candi.at[pl.ds(wid * 256, 256)])


@functools.partial(
    pl.kernel,
    out_type=jax.ShapeDtypeStruct((256,), I32),
    mesh=_mesh,
    compiler_params=pltpu.CompilerParams(
        needs_layout_passes=False, use_tc_tiling_on_sc=False),
    scratch_types=dict(
        vbuf=pltpu.VMEM((NW * 256,), F32),
        ibuf=pltpu.VMEM((NW * 256,), I32),
        hist=pltpu.VMEM((4096,), I32),
        hsum=pltpu.VMEM((256,), I32),
        gt_v=pltpu.VMEM((272,), F32),
        gt_i=pltpu.VMEM((272,), I32),
        eq_i=pltpu.VMEM((272,), I32),
        out_v=pltpu.VMEM((256,), F32),
        out_i=pltpu.VMEM((256,), I32),
    ),
)
def _k3(candv, candi, topk, vbuf, ibuf, hist, hsum, gt_v, gt_i, eq_i, out_v,
        out_i):
    wid = _wid()

    @pl.when(wid == 0)
    def _run():
        pltpu.sync_copy(candv, vbuf)
        pltpu.sync_copy(candi, ibuf)
        vbits, m = _radix_select(vbuf, NW * 16, TOPK, hist, hsum)
        _select_topk(vbuf, ibuf, NW * 16, 0, vbits, m, gt_v, gt_i, eq_i,
                     out_v, out_i)
        pltpu.sync_copy(out_i, topk)


@functools.partial(
    pl.kernel,
    out_type=jax.ShapeDtypeStruct((NENT, D), F32),
    mesh=_mesh,
    compiler_params=pltpu.CompilerParams(
        needs_layout_passes=False, use_tc_tiling_on_sc=False),
    scratch_types=dict(
        rel_v=pltpu.VMEM((576,), F32),
        topk_v=pltpu.VMEM((256,), I32),
        hb=pltpu.VMEM((2, 128), I32),
        tb=pltpu.VMEM((2, 128), I32),
        yb=pltpu.VMEM((2, 128), I32),
        ab=pltpu.VMEM((2, 128), F32),
        rows=pltpu.VMEM((256, D), F32),
        sidx=pltpu.VMEM((2, 128), I32),
        acc_sh=pltpu.VMEM_SHARED((25024, D), F32),
        s1=pltpu.SemaphoreType.DMA,
    ),
)
def _k4(ent, usr, relf, head2, tail2, typ2, attn2, topk, uiu2, uii2, uiw2,
        ent_out, rel_v, topk_v, hb, tb, yb, ab, rows, sidx, acc_sh, s1):
    c = lax.axis_index("c")
    s = lax.axis_index("s")
    it = _it()
    zf = jnp.zeros((LN,), F32)

    @pl.loop(0, 1024)
    def _zm(i):
        pos = i * 16 + it
        plsc.store_scatter(rows, [pos >> 6, pos & 63], zf)

    zrow = s * 1564
    for zo in range(6):
        pltpu.sync_copy(rows, acc_sh.at[pl.ds(zrow + zo * 256, 256)])
    pltpu.sync_copy(rows.at[pl.ds(0, 28)], acc_sh.at[pl.ds(zrow + 1536, 28)])
    pltpu.sync_copy(relf, rel_v)
    pltpu.sync_copy(topk, topk_v)
    plsc.subcore_barrier()

    row_lo = c * 25000
    tbase = s * 400

    @pl.loop(0, 200)
    def _chunk(ci):
        r0 = tbase + ci * 2
        gidb = r0 * 128
        pltpu.sync_copy(head2.at[pl.ds(r0, 2)], hb)
        pltpu.sync_copy(tail2.at[pl.ds(r0, 2)], tb)
        pltpu.sync_copy(typ2.at[pl.ds(r0, 2)], yb)
        pltpu.sync_copy(attn2.at[pl.ds(r0, 2)], ab)
        for tg in range(16):
            tk = topk_v[pl.ds(tg * 16, 16)]
            loc = tk - gidb
            mk = (loc >= 0) & (loc < 256)
            locc = jnp.where(mk, loc, 0)
            plsc.store_scatter(ab, [locc >> 7, locc & 127], zf, mask=mk)
        cps = [pltpu.async_copy(ent.at[tb.at[j]],
                                rows.at[pl.ds(j * 128, 128)], s1)
               for j in range(2)]
        for cp in cps:
            cp.wait()

        @pl.loop(0, 16)
        def _g(g):
            pos = g * 16 + it
            rr = pos >> 7
            cc = pos & 127
            av = plsc.load_gather(ab, [rr, cc])
            tyv = plsc.load_gather(yb, [rr, cc])
            relbase = (tyv - 1) * 64
            for d in range(D):
                dv = jnp.full((LN,), d, I32)
                t = plsc.load_gather(rows, [pos, dv])
                r = plsc.load_gather(rel_v, [relbase + d])
                plsc.store_scatter(rows, [pos, dv], t * r * av)
            hv = plsc.load_gather(hb, [rr, cc])
            li = hv - row_lo
            ok = (li >= 0) & (li < 25000)
            plsc.store_scatter(sidx, [rr, cc], jnp.where(ok, li, 25000))

        for j in range(2):
            pltpu.sync_copy(rows.at[pl.ds(j * 128, 128)],
                            acc_sh.at[sidx.at[j]], add=True)

    @pl.when(c == 0)
    def _ui():
        @pl.loop(0, 76)
        def _uc(ci):
            r0 = s * 152 + ci * 2
            pltpu.sync_copy(uiu2.at[pl.ds(r0, 2)], hb)
            pltpu.sync_copy(uii2.at[pl.ds(r0, 2)], tb)
            pltpu.sync_copy(uiw2.at[pl.ds(r0, 2)], ab)
            cps = [pltpu.async_copy(usr.at[hb.at[j]],
                                    rows.at[pl.ds(j * 128, 128)], s1)
                   for j in range(2)]
            for cp in cps:
                cp.wait()

            @pl.loop(0, 16)
            def _g(g):
                pos = g * 16 + it
                rr = pos >> 7
                cc = pos & 127
                wv = plsc.load_gather(ab, [rr, cc])
                for d in range(D):
                    dv = jnp.full((LN,), d, I32)
                    u = plsc.load_gather(rows, [pos, dv])
                    plsc.store_scatter(rows, [pos, dv], u * wv)
                iv = plsc.load_gather(tb, [rr, cc])
                plsc.store_scatter(sidx, [rr, cc], iv)

            for j in range(2):
                pltpu.sync_copy(rows.at[pl.ds(j * 128, 128)],
                                acc_sh.at[sidx.at[j]], add=True)

    plsc.subcore_barrier()

    @pl.when(s == 0)
    def _out():
        pltpu.sync_copy(acc_sh.at[pl.ds(0, 25000)],
                        ent_out.at[pl.ds(row_lo, 25000)])


@functools.partial(
    pl.kernel,
    out_type=jax.ShapeDtypeStruct((NU, D), F32),
    mesh=_mesh,
    compiler_params=pltpu.CompilerParams(
        needs_layout_passes=False, use_tc_tiling_on_sc=False),
    scratch_types=dict(
        ub=pltpu.VMEM((4, 128), I32),
        ib=pltpu.VMEM((4, 128), I32),
        wb=pltpu.VMEM((4, 128), F32),
        rows=pltpu.VMEM((512, D), F32),
        sidx=pltpu.VMEM((4, 128), I32),
        acc_sh=pltpu.VMEM_SHARED((15008, D), F32),
        s1=pltpu.SemaphoreType.DMA,
    ),
)
def _k5(ent, uiu2, uii2, uiw2, usr_out, ub, ib, wb, rows, sidx, acc_sh, s1):
    c = lax.axis_index("c")
    s = lax.axis_index("s")
    it = _it()
    zf = jnp.zeros((LN,), F32)

    @pl.loop(0, 2048)
    def _zm(i):
        pos = i * 16 + it
        plsc.store_scatter(rows, [pos >> 6, pos & 63], zf)

    zrow = s * 938
    pltpu.sync_copy(rows, acc_sh.at[pl.ds(zrow, 512)])
    pltpu.sync_copy(rows.at[pl.ds(0, 426)], acc_sh.at[pl.ds(zrow + 512, 426)])
    plsc.subcore_barrier()

    row_lo = c * 15000

    @pl.loop(0, 38)
    def _uc(ci):
        r0 = s * 152 + ci * 4
        pltpu.sync_copy(uiu2.at[pl.ds(r0, 4)], ub)
        pltpu.sync_copy(uii2.at[pl.ds(r0, 4)], ib)
        pltpu.sync_copy(uiw2.at[pl.ds(r0, 4)], wb)
        cps = [pltpu.async_copy(ent.at[ib.at[j]],
                                rows.at[pl.ds(j * 128, 128)], s1)
               for j in range(4)]
        for cp in cps:
            cp.wait()

        @pl.loop(0, 32)
        def _g(g):
            pos = g * 16 + it
            rr = pos >> 7
            cc = pos & 127
            wv = plsc.load_gather(wb, [rr, cc])
            for d in range(D):
                dv = jnp.full((LN,), d, I32)
                x = plsc.load_gather(rows, [pos, dv])
                plsc.store_scatter(rows, [pos, dv], x * wv)
            uv = plsc.load_gather(ub, [rr, cc])
            li = uv - row_lo
            ok = (li >= 0) & (li < 15000)
            plsc.store_scatter(sidx, [rr, cc], jnp.where(ok, li, 15000))

        for j in range(4):
            pltpu.sync_copy(rows.at[pl.ds(j * 128, 128)],
                            acc_sh.at[sidx.at[j]], add=True)

    plsc.subcore_barrier()

    @pl.when(s == 0)
    def _out():
        pltpu.sync_copy(acc_sh.at[pl.ds(0, 15000)],
                        usr_out.at[pl.ds(row_lo, 15000)])


@functools.partial(
    pl.kernel,
    out_type=[
        jax.ShapeDtypeStruct((BATCH,), F32),
        jax.ShapeDtypeStruct((BATCH,), F32),
        jax.ShapeDtypeStruct((BATCH,), F32),
        jax.ShapeDtypeStruct((TOPK,), F32),
        jax.ShapeDtypeStruct((TOPK,), F32),
        jax.ShapeDtypeStruct((TOPK,), F32),
    ],
    mesh=_mesh,
    compiler_params=pltpu.CompilerParams(
        needs_layout_passes=False, use_tc_tiling_on_sc=False),
    scratch_types=dict(
        iu=pltpu.VMEM((32,), I32),
        ip=pltpu.VMEM((32,), I32),
        iq=pltpu.VMEM((32,), I32),
        ru0=pltpu.VMEM((32, D), F32), ru1=pltpu.VMEM((32, D), F32),
        ru2=pltpu.VMEM((32, D), F32),
        rp0=pltpu.VMEM((32, D), F32), rp1=pltpu.VMEM((32, D), F32),
        rp2=pltpu.VMEM((32, D), F32),
        rq0=pltpu.VMEM((32, D), F32), rq1=pltpu.VMEM((32, D), F32),
        rq2=pltpu.VMEM((32, D), F32),
        sposb=pltpu.VMEM((32,), F32),
        snegb=pltpu.VMEM((32,), F32),
        ssqb=pltpu.VMEM((32,), F32),
        tk16=pltpu.VMEM((16,), I32),
        hsel=pltpu.VMEM((16,), I32),
        tsel=pltpu.VMEM((16,), I32),
        ysel=pltpu.VMEM((16,), I32),
        ysb=pltpu.VMEM((16,), I32),
        mh0=pltpu.VMEM((16, D), F32), mh1=pltpu.VMEM((16, D), F32),
        mh2=pltpu.VMEM((16, D), F32),
        mt0=pltpu.VMEM((16, D), F32), mt1=pltpu.VMEM((16, D), F32),
        mt2=pltpu.VMEM((16, D), F32),
        mr=pltpu.VMEM((16, D), F32),
        mnumb=pltpu.VMEM((16,), F32),
        mnpb=pltpu.VMEM((16,), F32),
        mntb=pltpu.VMEM((16,), F32),
        sm=pltpu.SemaphoreType.DMA,
    ),
)
def _k8(usr0, usr1, usr2, ent0, ent1, ent2, rel2, head1, tail1, typ1, topk,
        users, pos_items, neg_items, spos, sneg, ssq, mnum, mnp, mnt,
        iu, ip, iq, ru0, ru1, ru2, rp0, rp1, rp2, rq0, rq1, rq2,
        sposb, snegb, ssqb, tk16, hsel, tsel, ysel, ysb,
        mh0, mh1, mh2, mt0, mt1, mt2, mr, mnumb, mnpb, mntb, sm):
    c = lax.axis_index("c")
    s = lax.axis_index("s")
    wid = c * NS + s
    it = _it()
    zf = jnp.zeros((LN,), F32)
    sb = wid * 32
    pltpu.sync_copy(users.at[pl.ds(sb, 32)], iu)
    pltpu.sync_copy(pos_items.at[pl.ds(sb, 32)], ip)
    pltpu.sync_copy(neg_items.at[pl.ds(sb, 32)], iq)
    for tbl, idx, dst in ((usr0, iu, ru0), (usr1, iu, ru1), (usr2, iu, ru2),
                          (ent0, ip, rp0), (ent1, ip, rp1), (ent2, ip, rp2),
                          (ent0, iq, rq0), (ent1, iq, rq1), (ent2, iq, rq2)):
        pltpu.async_copy(tbl.at[idx], dst, sm).wait()

    @pl.loop(0, 2)
    def _sg(si):
        rowv = si * 16 + it
        sp = zf
        sn = zf
        sq = zf
        for d in range(D):
            dv = jnp.full((LN,), d, I32)
            u = (plsc.load_gather(ru0, [rowv, dv])
                 + plsc.load_gather(ru1, [rowv, dv])
                 + plsc.load_gather(ru2, [rowv, dv]))
            p = (plsc.load_gather(rp0, [rowv, dv])
                 + plsc.load_gather(rp1, [rowv, dv])
                 + plsc.load_gather(rp2, [rowv, dv]))
            q = (plsc.load_gather(rq0, [rowv, dv])
                 + plsc.load_gather(rq1, [rowv, dv])
                 + plsc.load_gather(rq2, [rowv, dv]))
            sp = sp + u * p
            sn = sn + u * q
            sq = sq + u * u + p * p + q * q
        sposb[pl.ds(si * 16, 16)] = sp
        snegb[pl.ds(si * 16, 16)] = sn
        ssqb[pl.ds(si * 16, 16)] = sq

    pltpu.sync_copy(sposb, spos.at[pl.ds(sb, 32)])
    pltpu.sync_copy(snegb, sneg.at[pl.ds(sb, 32)])
    pltpu.sync_copy(ssqb, ssq.at[pl.ds(sb, 32)])

    @pl.when(c == 0)
    def _mae():
        eb = s * 16
        pltpu.sync_copy(topk.at[pl.ds(eb, 16)], tk16)
        pltpu.async_copy(head1.at[tk16], hsel, sm).wait()
        pltpu.async_copy(tail1.at[tk16], tsel, sm).wait()
        pltpu.async_copy(typ1.at[tk16], ysel, sm).wait()
        ysb[...] = ysel[...] - 1
        for tbl, idx, dst in ((ent0, hsel, mh0), (ent1, hsel, mh1),
                              (ent2, hsel, mh2), (ent0, tsel, mt0),
                              (ent1, tsel, mt1), (ent2, tsel, mt2)):
            pltpu.async_copy(tbl.at[idx], dst, sm).wait()
        pltpu.async_copy(rel2.at[ysb], mr, sm).wait()
        nm = zf
        npv = zf
        ntv = zf
        for d in range(D):
            dv = jnp.full((LN,), d, I32)
            mh = (plsc.load_gather(mh0, [it, dv])
                  + plsc.load_gather(mh1, [it, dv])
                  + plsc.load_gather(mh2, [it, dv]))
            mt = (plsc.load_gather(mt0, [it, dv])
                  + plsc.load_gather(mt1, [it, dv])
                  + plsc.load_gather(mt2, [it, dv]))
            r = plsc.load_gather(mr, [it, dv])
            pr = mh * r
            nm = nm + pr * mt
            npv = npv + pr * pr
            ntv = ntv + mt * mt
        mnumb[...] = nm
        mnpb[...] = npv
        mntb[...] = ntv
        pltpu.sync_copy(mnumb, mnum.at[pl.ds(eb, 16)])
        pltpu.sync_copy(mnpb, mnp.at[pl.ds(eb, 16)])
        pltpu.sync_copy(mntb, mnt.at[pl.ds(eb, 16)])


def _k9(spos, sneg, ssq, mnum, mnp, mnt):
    def body(sp, sn, sq, nm, npr, ntr, o):
        x = sn[...] - sp[...]
        softp = jnp.maximum(x, 0.0) + jnp.log1p(jnp.exp(-jnp.abs(x)))
        rec = jnp.sum(softp) / BATCH
        reg = 1e-5 * jnp.sum(sq[...]) / (2.0 * BATCH)
        cos = nm[...] / (jnp.sqrt(npr[...]) * jnp.sqrt(ntr[...]) + 1e-8)
        mae = jnp.sum(1.0 - cos) / TOPK
        o[...] = jnp.reshape(rec + reg + 0.1 * mae, (1, 1))

    return pl.pallas_call(
        body, out_shape=jax.ShapeDtypeStruct((1, 1), F32))(
            spos.reshape(8, 128), sneg.reshape(8, 128), ssq.reshape(8, 128),
            mnum.reshape(2, 128), mnp.reshape(2, 128), mnt.reshape(2, 128))


def _pad1(x, n, val):
    return jnp.concatenate([x, jnp.full((n - x.shape[0],), val, x.dtype)])


def kernel(all_embed, relation_emb, inter_edge_w, edge_index, edge_type,
           inter_edge, users, pos_items, neg_items):
    ae = all_embed.astype(F32)
    usr0 = ae[:NU]
    ent0 = ae[NU:]
    relf = relation_emb.astype(F32).reshape(-1)
    head1 = _pad1(edge_index[0].astype(I32), EP, 0)
    tail1 = _pad1(edge_index[1].astype(I32), EP, 0)
    typ1 = _pad1(edge_type.astype(I32), EP, 1)
    head2 = head1.reshape(ER, 128)
    tail2 = tail1.reshape(ER, 128)
    typ2 = typ1.reshape(ER, 128)
    uiu2 = _pad1(inter_edge[0].astype(I32), UIP, 0).reshape(UIR, 128)
    uii2 = _pad1(inter_edge[1].astype(I32), UIP, 0).reshape(UIR, 128)
    uiw2 = _pad1(inter_edge_w.astype(F32), UIP, 0.0).reshape(UIR, 128)

    ex2, den_p = _k1(ent0, relf, head2, tail2, typ2)
    attn2, candv, candi = _k2(ex2, head2, den_p)
    topk = _k3(candv, candi)
    ent1 = _k4(ent0, usr0, relf, head2, tail2, typ2, attn2, topk, uiu2, uii2,
               uiw2)
    usr1 = _k5(ent0, uiu2, uii2, uiw2)
    ent2 = _k4(ent1, usr1, relf, head2, tail2, typ2, attn2, topk, uiu2, uii2,
               uiw2)
    usr2 = _k5(ent1, uiu2, uii2, uiw2)
    spos, sneg, ssq, mnum, mnp, mnt = _k8(
        usr0, usr1, usr2, ent0, ent1, ent2, relation_emb.astype(F32), head1,
        tail1, typ1, topk, users.astype(I32), pos_items.astype(I32),
        neg_items.astype(I32))
    return _k9(spos, sneg, ssq, mnum, mnp, mnt).reshape(())
